# baseline (speedup 1.0000x reference)
import jax
import jax.numpy as jnp
from jax import lax
from jax.experimental import pallas as pl
from jax.experimental.pallas import tpu as pltpu
from jax.experimental.pallas import tpu_sc as plsc

_DIM = 64
_SEQ = 200
_EPS = 1e-12
_NC = 2
_NS = 16
_NW = _NC * _NS
_LANES = 16
_NVR = _DIM // _LANES
_CSEQ = 2
_CTOK = _CSEQ * _SEQ
_IDXW = 100


def _rsqrt(x):
    i = lax.bitcast_convert_type(x, jnp.int32)
    i = jnp.int32(0x5F3759DF) - lax.shift_right_logical(i, 1)
    y = lax.bitcast_convert_type(i, jnp.float32)
    for _ in range(3):
        y = y * (1.5 - 0.5 * x * y * y)
    return y


def _body(x_hbm, tok_hbm, pos_hbm, out_hbm,
          idx0, idx1, rows0, rows1, pos_v,
          gsem0, gsem1, osem0, osem1):
    cid = lax.axis_index("c")
    sid = lax.axis_index("s")
    wid = sid * _NC + cid
    n_chunks = x_hbm.shape[0] // (_NW * (_CTOK // _IDXW))
    chunk0 = wid * n_chunks

    for s in range(_CSEQ):
        pltpu.sync_copy(pos_hbm, pos_v.at[pl.ds(s * _SEQ, _SEQ)])

    def start_fetch(k, idx_v, rows_v, sem):
        base = (chunk0 + k) * (_CTOK // _IDXW)
        pltpu.sync_copy(x_hbm.at[pl.ds(base, _CTOK // _IDXW)], idx_v)
        for h in range(_CTOK // _IDXW):
            pltpu.async_copy(tok_hbm.at[idx_v.at[h]],
                             rows_v.at[pl.ds(h * _IDXW, _IDXW)], sem)

    def wait_fetch(rows_v, sem):
        for h in range(_CTOK // _IDXW):
            pltpu.make_async_copy(tok_hbm.at[pl.ds(0, _IDXW)],
                                  rows_v.at[pl.ds(h * _IDXW, _IDXW)],
                                  sem).wait()

    def compute(rows_v):
        @plsc.parallel_loop(0, _CTOK, step=1, unroll=8)
        def tok_body(i):
            e = [rows_v[i, pl.ds(d * _LANES, _LANES)]
                 + pos_v[i, pl.ds(d * _LANES, _LANES)]
                 for d in range(_NVR)]
            s = (e[0] + e[1]) + (e[2] + e[3])
            q = (e[0] * e[0] + e[1] * e[1]) + (e[2] * e[2] + e[3] * e[3])
            m = jnp.sum(s) * (1.0 / _DIM)
            var = jnp.sum(q) * (1.0 / _DIM) - m * m
            inv = _rsqrt(var + _EPS)
            for d in range(_NVR):
                rows_v[i, pl.ds(d * _LANES, _LANES)] = (e[d] - m) * inv

    def start_flush(k, rows_v, sem):
        pltpu.async_copy(rows_v, out_hbm.at[pl.ds((chunk0 + k) * _CTOK,
                                                  _CTOK)], sem)

    def wait_flush(rows_v, sem):
        pltpu.make_async_copy(rows_v, out_hbm.at[pl.ds(0, _CTOK)],
                              sem).wait()

    bufs = ((idx0, rows0, gsem0, osem0), (idx1, rows1, gsem1, osem1))

    start_fetch(0, *bufs[0][:3])
    start_fetch(1, *bufs[1][:3])
    wait_fetch(rows0, gsem0)
    compute(rows0)
    start_flush(0, rows0, osem0)

    def step(t, carry):
        k = 2 * t + 1
        for (idx_v, rows_v, gsem, osem), other, kk in (
                (bufs[1], bufs[0], k), (bufs[0], bufs[1], k + 1)):
            wait_flush(other[1], other[3])
            start_fetch(kk + 1, *other[:3])
            wait_fetch(rows_v, gsem)
            compute(rows_v)
            start_flush(kk, rows_v, osem)
        return carry

    lax.fori_loop(0, (n_chunks - 2) // 2, step, 0)

    wait_flush(rows0, osem0)
    wait_fetch(rows1, gsem1)
    compute(rows1)
    start_flush(n_chunks - 1, rows1, osem1)
    wait_flush(rows1, osem1)


@jax.jit
def kernel(x, tok_table, pos_table, gamma, beta):
    batch, seq = x.shape
    dim = tok_table.shape[1]
    x2 = x.reshape(batch * seq // _IDXW, _IDXW)
    pos = pos_table[:seq]
    mesh = plsc.VectorSubcoreMesh(core_axis_name="c", subcore_axis_name="s")
    h = pl.kernel(
        _body,
        out_type=jax.ShapeDtypeStruct((batch * seq, dim), jnp.float32),
        mesh=mesh,
        scratch_types=[
            pltpu.VMEM((_CTOK // _IDXW, _IDXW), jnp.int32),
            pltpu.VMEM((_CTOK // _IDXW, _IDXW), jnp.int32),
            pltpu.VMEM((_CTOK, dim), jnp.float32),
            pltpu.VMEM((_CTOK, dim), jnp.float32),
            pltpu.VMEM((_CTOK, dim), jnp.float32),
            pltpu.SemaphoreType.DMA,
            pltpu.SemaphoreType.DMA,
            pltpu.SemaphoreType.DMA,
            pltpu.SemaphoreType.DMA,
        ],
        compiler_params=pltpu.CompilerParams(
            needs_layout_passes=False, use_tc_tiling_on_sc=False),
    )(x2, tok_table, pos)
    return (gamma * h + beta).reshape(batch, seq, dim)


# speedup vs baseline: 1.1508x; 1.1508x over previous
import jax
import jax.numpy as jnp
from jax import lax
from jax.experimental import pallas as pl
from jax.experimental.pallas import tpu as pltpu
from jax.experimental.pallas import tpu_sc as plsc

_DIM = 64
_SEQ = 200
_EPS = 1e-12
_NC = 2
_NS = 16
_NW = _NC * _NS
_LANES = 16
_NVR = _DIM // _LANES
_CSEQ = 2
_CTOK = _CSEQ * _SEQ
_IDXW = 100


def _rsqrt(x):
    i = lax.bitcast_convert_type(x, jnp.int32)
    i = jnp.int32(0x5F3759DF) - lax.shift_right_logical(i, 1)
    y = lax.bitcast_convert_type(i, jnp.float32)
    for _ in range(3):
        y = y * (1.5 - 0.5 * x * y * y)
    return y


def _body(x_hbm, tok_hbm, pos_hbm, gb_hbm, out_hbm,
          idx0, idx1, rows0, rows1, pos_v, gb_v,
          gsem0, gsem1, osem0, osem1):
    cid = lax.axis_index("c")
    sid = lax.axis_index("s")
    wid = sid * _NC + cid
    n_chunks = x_hbm.shape[0] // (_NW * (_CTOK // _IDXW))
    chunk0 = wid * n_chunks

    for s in range(_CSEQ):
        pltpu.sync_copy(pos_hbm, pos_v.at[pl.ds(s * _SEQ, _SEQ)])
    pltpu.sync_copy(gb_hbm, gb_v)

    def start_fetch(k, idx_v, rows_v, sem):
        base = (chunk0 + k) * (_CTOK // _IDXW)
        pltpu.sync_copy(x_hbm.at[pl.ds(base, _CTOK // _IDXW)], idx_v)
        for h in range(_CTOK // _IDXW):
            pltpu.async_copy(tok_hbm.at[idx_v.at[h]],
                             rows_v.at[pl.ds(h * _IDXW, _IDXW)], sem)

    def wait_fetch(rows_v, sem):
        for h in range(_CTOK // _IDXW):
            pltpu.make_async_copy(tok_hbm.at[pl.ds(0, _IDXW)],
                                  rows_v.at[pl.ds(h * _IDXW, _IDXW)],
                                  sem).wait()

    def compute(rows_v):
        @plsc.parallel_loop(0, _CTOK, step=1, unroll=8)
        def tok_body(i):
            e = [rows_v[i, pl.ds(d * _LANES, _LANES)]
                 + pos_v[i, pl.ds(d * _LANES, _LANES)]
                 for d in range(_NVR)]
            s = (e[0] + e[1]) + (e[2] + e[3])
            q = (e[0] * e[0] + e[1] * e[1]) + (e[2] * e[2] + e[3] * e[3])
            m = jnp.sum(s) * (1.0 / _DIM)
            var = jnp.sum(q) * (1.0 / _DIM) - m * m
            inv = _rsqrt(var + _EPS)
            for d in range(_NVR):
                g = gb_v[0, pl.ds(d * _LANES, _LANES)]
                b = gb_v[1, pl.ds(d * _LANES, _LANES)]
                rows_v[i, pl.ds(d * _LANES, _LANES)] = \
                    (e[d] - m) * (inv * g) + b

    def start_flush(k, rows_v, sem):
        pltpu.async_copy(rows_v, out_hbm.at[pl.ds((chunk0 + k) * _CTOK,
                                                  _CTOK)], sem)

    def wait_flush(rows_v, sem):
        pltpu.make_async_copy(rows_v, out_hbm.at[pl.ds(0, _CTOK)],
                              sem).wait()

    bufs = ((idx0, rows0, gsem0, osem0), (idx1, rows1, gsem1, osem1))

    start_fetch(0, *bufs[0][:3])
    start_fetch(1, *bufs[1][:3])
    wait_fetch(rows0, gsem0)
    compute(rows0)
    start_flush(0, rows0, osem0)

    def step(t, carry):
        k = 2 * t + 1
        for (idx_v, rows_v, gsem, osem), other, kk in (
                (bufs[1], bufs[0], k), (bufs[0], bufs[1], k + 1)):
            wait_flush(other[1], other[3])
            start_fetch(kk + 1, *other[:3])
            wait_fetch(rows_v, gsem)
            compute(rows_v)
            start_flush(kk, rows_v, osem)
        return carry

    lax.fori_loop(0, (n_chunks - 2) // 2, step, 0)

    wait_flush(rows0, osem0)
    wait_fetch(rows1, gsem1)
    compute(rows1)
    start_flush(n_chunks - 1, rows1, osem1)
    wait_flush(rows1, osem1)


@jax.jit
def kernel(x, tok_table, pos_table, gamma, beta):
    batch, seq = x.shape
    dim = tok_table.shape[1]
    x2 = x.reshape(batch * seq // _IDXW, _IDXW)
    pos = pos_table[:seq]
    gb = jnp.stack([gamma, beta])
    mesh = plsc.VectorSubcoreMesh(core_axis_name="c", subcore_axis_name="s")
    h = pl.kernel(
        _body,
        out_type=jax.ShapeDtypeStruct((batch * seq, dim), jnp.float32),
        mesh=mesh,
        scratch_types=[
            pltpu.VMEM((_CTOK // _IDXW, _IDXW), jnp.int32),
            pltpu.VMEM((_CTOK // _IDXW, _IDXW), jnp.int32),
            pltpu.VMEM((_CTOK, dim), jnp.float32),
            pltpu.VMEM((_CTOK, dim), jnp.float32),
            pltpu.VMEM((_CTOK, dim), jnp.float32),
            pltpu.VMEM((2, dim), jnp.float32),
            pltpu.SemaphoreType.DMA,
            pltpu.SemaphoreType.DMA,
            pltpu.SemaphoreType.DMA,
            pltpu.SemaphoreType.DMA,
        ],
        compiler_params=pltpu.CompilerParams(
            needs_layout_passes=False, use_tc_tiling_on_sc=False),
    )(x2, tok_table, pos, gb)
    return h.reshape(batch, seq, dim)
